# baseline (speedup 1.0000x reference)
import functools

import jax
import jax.numpy as jnp
from jax import lax
from jax.experimental import pallas as pl
from jax.experimental.pallas import tpu as pltpu
from jax.experimental.pallas import tpu_sc as plsc

_pcall = functools.partial(pl.pallas_call)

_E = 8
_K = 2
_NEG = -3.0e38
_NC, _NS = 2, 16
_NW = _NC * _NS
_TG = 256
_KB = 512
_CH = 32


def _gate_body(x_ref, gw_ref, gb_ref, eidx_ref, gsc_ref):
    x = x_ref[...]
    logits = lax.dot_general(x, gw_ref[...], (((1,), (1,)), ((), ())),
                             preferred_element_type=jnp.float32)
    logits = logits + gb_ref[...]
    iota = lax.broadcasted_iota(jnp.int32, logits.shape, 1)
    m1 = jnp.max(logits, axis=1, keepdims=True)
    i1 = jnp.min(jnp.where(logits == m1, iota, _E), axis=1, keepdims=True)
    l2 = jnp.where(iota == i1, _NEG, logits)
    m2 = jnp.max(l2, axis=1, keepdims=True)
    i2 = jnp.min(jnp.where(l2 == m2, iota, _E), axis=1, keepdims=True)
    p2 = jnp.exp(m2 - m1)
    denom = 1.0 + p2
    eidx_ref[...] = jnp.concatenate([i1, i2], axis=1)
    gsc_ref[...] = jnp.concatenate([1.0 / denom, p2 / denom], axis=1)


def _gating(flat, gate_w, gate_b):
    n, _ = flat.shape
    return _pcall(
        _gate_body,
        out_shape=(jax.ShapeDtypeStruct((n, _K), jnp.int32),
                   jax.ShapeDtypeStruct((n, _K), jnp.float32)),
    )(flat, gate_w, gate_b.reshape(1, _E))


def _make_dispatch(n, d, p):
    nk = n * _K
    nch = nk // _NW // _CH
    mesh = plsc.VectorSubcoreMesh(core_axis_name="c", subcore_axis_name="s")

    @functools.partial(
        pl.kernel, mesh=mesh,
        out_type=jax.ShapeDtypeStruct((p, d), jnp.float32),
        scratch_types=[
            pltpu.VMEM((_CH,), jnp.int32),
            pltpu.VMEM((_CH,), jnp.int32),
            pltpu.VMEM((_CH, d), jnp.float32),
            pltpu.SemaphoreType.DMA,
        ],
    )
    def dispatch(flat_hbm, tok_hbm, dest_hbm, xg_hbm, tok_v, dst_v, rows_v, sem):
        wid = lax.axis_index("s") * _NC + lax.axis_index("c")

        def body(ci, carry):
            row = wid * nch + ci
            pltpu.sync_copy(tok_hbm.at[row], tok_v)
            pltpu.sync_copy(dest_hbm.at[row], dst_v)
            pltpu.async_copy(flat_hbm.at[tok_v], rows_v, sem).wait()
            pltpu.async_copy(rows_v, xg_hbm.at[dst_v], sem).wait()
            return carry

        lax.fori_loop(0, nch, body, 0)

    return dispatch


def _gmm_body(eid_ref, vld_ref, xg_ref, w1_ref, w2_ref, y_ref):
    i = pl.program_id(0)
    j = pl.program_id(1)

    @pl.when(vld_ref[i] == 1)
    def _():
        x = xg_ref[...].astype(jnp.bfloat16)
        h = lax.dot_general(x, w1_ref[0], (((1,), (1,)), ((), ())),
                            preferred_element_type=jnp.float32)
        h = 0.5 * h * (1.0 + lax.erf(h * 0.7071067811865476))
        y = lax.dot_general(h.astype(jnp.bfloat16), w2_ref[0],
                            (((1,), (1,)), ((), ())),
                            preferred_element_type=jnp.float32)

        @pl.when(j == 0)
        def _():
            y_ref[...] = y

        @pl.when(j > 0)
        def _():
            y_ref[...] += y


def _gmm(eid_tile, valid, xg, W1, W2):
    p, d = xg.shape
    _, dh, _ = W1.shape
    grid_spec = pltpu.PrefetchScalarGridSpec(
        num_scalar_prefetch=2,
        grid=(p // _TG, dh // _KB),
        in_specs=[
            pl.BlockSpec((_TG, d), lambda i, j, eid, vld: (i, 0)),
            pl.BlockSpec((1, _KB, d), lambda i, j, eid, vld: (eid[i], j, 0)),
            pl.BlockSpec((1, d, _KB), lambda i, j, eid, vld: (eid[i], 0, j)),
        ],
        out_specs=pl.BlockSpec((_TG, d), lambda i, j, eid, vld: (i, 0)),
    )
    return _pcall(
        _gmm_body,
        grid_spec=grid_spec,
        out_shape=jax.ShapeDtypeStruct((p, d), jnp.float32),
    )(eid_tile, valid, xg, W1, W2)


def _make_gather2(n, d, p):
    nch = n // _NW // _CH
    mesh = plsc.VectorSubcoreMesh(core_axis_name="c", subcore_axis_name="s")

    @functools.partial(
        pl.kernel, mesh=mesh,
        out_type=(jax.ShapeDtypeStruct((n, d), jnp.float32),
                  jax.ShapeDtypeStruct((n, d), jnp.float32)),
        scratch_types=[
            pltpu.VMEM((_CH,), jnp.int32),
            pltpu.VMEM((_CH, d), jnp.float32),
            pltpu.SemaphoreType.DMA,
        ],
    )
    def gather2(y_hbm, p0_hbm, p1_hbm, y0_hbm, y1_hbm, idx_v, rows_v, sem):
        wid = lax.axis_index("s") * _NC + lax.axis_index("c")

        def body(ci, carry):
            row = wid * nch + ci
            off = pl.multiple_of(row * _CH, _CH)
            pltpu.sync_copy(p0_hbm.at[row], idx_v)
            pltpu.async_copy(y_hbm.at[idx_v], rows_v, sem).wait()
            pltpu.sync_copy(rows_v, y0_hbm.at[pl.ds(off, _CH)])
            pltpu.sync_copy(p1_hbm.at[row], idx_v)
            pltpu.async_copy(y_hbm.at[idx_v], rows_v, sem).wait()
            pltpu.sync_copy(rows_v, y1_hbm.at[pl.ds(off, _CH)])
            return carry

        lax.fori_loop(0, nch, body, 0)

    return gather2


def _ln_body(y0_ref, y1_ref, g0_ref, g1_ref, res_ref, lng_ref, lnb_ref, o_ref):
    a = g0_ref[...] * y0_ref[...] + g1_ref[...] * y1_ref[...] + res_ref[...]
    mu = jnp.mean(a, axis=1, keepdims=True)
    var = jnp.mean((a - mu) ** 2, axis=1, keepdims=True)
    o_ref[...] = (a - mu) * lax.rsqrt(var + 1e-5) * lng_ref[...] + lnb_ref[...]


def _ln(y0, y1, g0, g1, flat, ln_g, ln_b, tb=512):
    n, d = flat.shape
    return _pcall(
        _ln_body,
        grid=(n // tb,),
        in_specs=[
            pl.BlockSpec((tb, d), lambda i: (i, 0)),
            pl.BlockSpec((tb, d), lambda i: (i, 0)),
            pl.BlockSpec((tb, 1), lambda i: (i, 0)),
            pl.BlockSpec((tb, 1), lambda i: (i, 0)),
            pl.BlockSpec((tb, d), lambda i: (i, 0)),
            pl.BlockSpec((1, d), lambda i: (0, 0)),
            pl.BlockSpec((1, d), lambda i: (0, 0)),
        ],
        out_specs=pl.BlockSpec((tb, d), lambda i: (i, 0)),
        out_shape=jax.ShapeDtypeStruct((n, d), jnp.float32),
    )(y0, y1, g0, g1, flat, ln_g.reshape(1, d), ln_b.reshape(1, d))


def kernel(inp, gate_w, gate_b, W1, W2, ln_g, ln_b, bias):
    s, b, d = inp.shape
    n = s * b
    nk = n * _K
    p = nk + _E * _TG

    flat = inp.reshape(n, d)
    eidx, gsc = _gating(flat, gate_w, gate_b)

    ef = eidx.reshape(nk)
    onehot = (ef[:, None] == jnp.arange(_E, dtype=jnp.int32)[None, :]
              ).astype(jnp.int32)
    cum = jnp.cumsum(onehot, axis=0)
    counts = cum[-1]
    padded = ((counts + _TG - 1) // _TG) * _TG
    pcum = jnp.cumsum(padded)
    pbase = jnp.concatenate([jnp.zeros((1,), jnp.int32), pcum[:-1]])
    within = jnp.sum(onehot * (cum - 1), axis=1)
    baseof = jnp.sum(onehot * pbase[None, :], axis=1)
    dest = baseof + within

    tile_ends = pcum // _TG
    t = jnp.arange(p // _TG, dtype=jnp.int32)
    eid_tile = jnp.searchsorted(tile_ends, t, side="right").astype(jnp.int32)
    valid = (eid_tile < _E).astype(jnp.int32)
    eid_tile = jnp.minimum(eid_tile, _E - 1)

    dest2 = dest.reshape(nk // _CH, _CH)
    tok2 = (jnp.arange(nk, dtype=jnp.int32) // _K).reshape(nk // _CH, _CH)
    p01 = dest.reshape(n, _K)
    p0_2d = p01[:, 0].reshape(n // _CH, _CH)
    p1_2d = p01[:, 1].reshape(n // _CH, _CH)

    xg = _make_dispatch(n, d, p)(flat, tok2, dest2)
    y = _gmm(eid_tile, valid, xg,
             W1.astype(jnp.bfloat16), W2.astype(jnp.bfloat16))
    y0, y1 = _make_gather2(n, d, p)(y, p0_2d, p1_2d)
    out = _ln(y0, y1, gsc[:, 0].reshape(n, 1), gsc[:, 1].reshape(n, 1),
              flat, ln_g, ln_b)
    return out.reshape(s, b, d), bias


# speedup vs baseline: 1.3090x; 1.3090x over previous
import functools

import jax
import jax.numpy as jnp
from jax import lax
from jax.experimental import pallas as pl
from jax.experimental.pallas import tpu as pltpu
from jax.experimental.pallas import tpu_sc as plsc

_pcall = functools.partial(pl.pallas_call)

_E = 8
_K = 2
_NEG = -3.0e38
_NC, _NS = 2, 16
_NW = _NC * _NS
_TG = 256
_KB = 512
_CH = 32


def _gate_body(x_ref, gw_ref, gb_ref, eidx_ref, gsc_ref):
    x = x_ref[...]
    logits = lax.dot_general(x, gw_ref[...], (((1,), (1,)), ((), ())),
                             preferred_element_type=jnp.float32)
    logits = logits + gb_ref[...]
    iota = lax.broadcasted_iota(jnp.int32, logits.shape, 1)
    m1 = jnp.max(logits, axis=1, keepdims=True)
    i1 = jnp.min(jnp.where(logits == m1, iota, _E), axis=1, keepdims=True)
    l2 = jnp.where(iota == i1, _NEG, logits)
    m2 = jnp.max(l2, axis=1, keepdims=True)
    i2 = jnp.min(jnp.where(l2 == m2, iota, _E), axis=1, keepdims=True)
    p2 = jnp.exp(m2 - m1)
    denom = 1.0 + p2
    eidx_ref[...] = jnp.concatenate([i1, i2], axis=1)
    gsc_ref[...] = jnp.concatenate([1.0 / denom, p2 / denom], axis=1)


def _gating(flat, gate_w, gate_b):
    n, _ = flat.shape
    return _pcall(
        _gate_body,
        out_shape=(jax.ShapeDtypeStruct((n, _K), jnp.int32),
                   jax.ShapeDtypeStruct((n, _K), jnp.float32)),
    )(flat, gate_w, gate_b.reshape(1, _E))


def _make_dispatch(n, d, p):
    nk = n * _K
    nch = nk // _NW // _CH
    mesh = plsc.VectorSubcoreMesh(core_axis_name="c", subcore_axis_name="s")

    @functools.partial(
        pl.kernel, mesh=mesh,
        out_type=jax.ShapeDtypeStruct((p, d), jnp.float32),
        scratch_types=[
            pltpu.VMEM((_CH,), jnp.int32),
            pltpu.VMEM((_CH,), jnp.int32),
            pltpu.VMEM((_CH, d), jnp.float32),
            pltpu.SemaphoreType.DMA,
        ],
    )
    def dispatch(flat_hbm, tok_hbm, dest_hbm, xg_hbm, tok_v, dst_v, rows_v, sem):
        wid = lax.axis_index("s") * _NC + lax.axis_index("c")

        def body(ci, carry):
            row = wid * nch + ci
            pltpu.sync_copy(tok_hbm.at[row], tok_v)
            pltpu.sync_copy(dest_hbm.at[row], dst_v)
            pltpu.async_copy(flat_hbm.at[tok_v], rows_v, sem).wait()
            pltpu.async_copy(rows_v, xg_hbm.at[dst_v], sem).wait()
            return carry

        lax.fori_loop(0, nch, body, 0)

    return dispatch


def _gmm_body(eid_ref, vld_ref, xg_ref, w1_ref, w2_ref, y_ref):
    j = pl.program_id(0)
    i = pl.program_id(1)

    @pl.when(vld_ref[i] == 1)
    def _():
        x = xg_ref[pl.ds(i * _TG, _TG), :]
        h = lax.dot_general(x, w1_ref[0], (((1,), (1,)), ((), ())),
                            preferred_element_type=jnp.float32)
        h = 0.5 * h * (1.0 + lax.erf(h * 0.7071067811865476))
        y = lax.dot_general(h, w2_ref[0], (((1,), (1,)), ((), ())),
                            preferred_element_type=jnp.float32)

        @pl.when(j == 0)
        def _():
            y_ref[pl.ds(i * _TG, _TG), :] = y

        @pl.when(j > 0)
        def _():
            y_ref[pl.ds(i * _TG, _TG), :] += y


def _gmm(eid_tile, valid, xg, W1, W2):
    p, d = xg.shape
    _, dh, _ = W1.shape
    grid_spec = pltpu.PrefetchScalarGridSpec(
        num_scalar_prefetch=2,
        grid=(dh // _KB, p // _TG),
        in_specs=[
            pl.BlockSpec((p, d), lambda j, i, eid, vld: (0, 0)),
            pl.BlockSpec((1, _KB, d), lambda j, i, eid, vld: (eid[i], j, 0)),
            pl.BlockSpec((1, d, _KB), lambda j, i, eid, vld: (eid[i], 0, j)),
        ],
        out_specs=pl.BlockSpec((p, d), lambda j, i, eid, vld: (0, 0)),
    )
    return _pcall(
        _gmm_body,
        grid_spec=grid_spec,
        out_shape=jax.ShapeDtypeStruct((p, d), jnp.float32),
    )(eid_tile, valid, xg, W1, W2)


def _make_gather2(n, d, p):
    nch = n // _NW // _CH
    mesh = plsc.VectorSubcoreMesh(core_axis_name="c", subcore_axis_name="s")

    @functools.partial(
        pl.kernel, mesh=mesh,
        out_type=(jax.ShapeDtypeStruct((n, d), jnp.float32),
                  jax.ShapeDtypeStruct((n, d), jnp.float32)),
        scratch_types=[
            pltpu.VMEM((_CH,), jnp.int32),
            pltpu.VMEM((_CH, d), jnp.float32),
            pltpu.SemaphoreType.DMA,
        ],
    )
    def gather2(y_hbm, p0_hbm, p1_hbm, y0_hbm, y1_hbm, idx_v, rows_v, sem):
        wid = lax.axis_index("s") * _NC + lax.axis_index("c")

        def body(ci, carry):
            row = wid * nch + ci
            off = pl.multiple_of(row * _CH, _CH)
            pltpu.sync_copy(p0_hbm.at[row], idx_v)
            pltpu.async_copy(y_hbm.at[idx_v], rows_v, sem).wait()
            pltpu.sync_copy(rows_v, y0_hbm.at[pl.ds(off, _CH)])
            pltpu.sync_copy(p1_hbm.at[row], idx_v)
            pltpu.async_copy(y_hbm.at[idx_v], rows_v, sem).wait()
            pltpu.sync_copy(rows_v, y1_hbm.at[pl.ds(off, _CH)])
            return carry

        lax.fori_loop(0, nch, body, 0)

    return gather2


def _ln_body(y0_ref, y1_ref, g0_ref, g1_ref, res_ref, lng_ref, lnb_ref, o_ref):
    a = g0_ref[...] * y0_ref[...] + g1_ref[...] * y1_ref[...] + res_ref[...]
    mu = jnp.mean(a, axis=1, keepdims=True)
    var = jnp.mean((a - mu) ** 2, axis=1, keepdims=True)
    o_ref[...] = (a - mu) * lax.rsqrt(var + 1e-5) * lng_ref[...] + lnb_ref[...]


def _ln(y0, y1, g0, g1, flat, ln_g, ln_b, tb=512):
    n, d = flat.shape
    return _pcall(
        _ln_body,
        grid=(n // tb,),
        in_specs=[
            pl.BlockSpec((tb, d), lambda i: (i, 0)),
            pl.BlockSpec((tb, d), lambda i: (i, 0)),
            pl.BlockSpec((tb, 1), lambda i: (i, 0)),
            pl.BlockSpec((tb, 1), lambda i: (i, 0)),
            pl.BlockSpec((tb, d), lambda i: (i, 0)),
            pl.BlockSpec((1, d), lambda i: (0, 0)),
            pl.BlockSpec((1, d), lambda i: (0, 0)),
        ],
        out_specs=pl.BlockSpec((tb, d), lambda i: (i, 0)),
        out_shape=jax.ShapeDtypeStruct((n, d), jnp.float32),
    )(y0, y1, g0, g1, flat, ln_g.reshape(1, d), ln_b.reshape(1, d))


def kernel(inp, gate_w, gate_b, W1, W2, ln_g, ln_b, bias):
    s, b, d = inp.shape
    n = s * b
    nk = n * _K
    p = nk + _E * _TG

    flat = inp.reshape(n, d)
    eidx, gsc = _gating(flat, gate_w, gate_b)

    ef = eidx.reshape(nk)
    onehot = (ef[:, None] == jnp.arange(_E, dtype=jnp.int32)[None, :]
              ).astype(jnp.int32)
    cum = jnp.cumsum(onehot, axis=0)
    counts = cum[-1]
    padded = ((counts + _TG - 1) // _TG) * _TG
    pcum = jnp.cumsum(padded)
    pbase = jnp.concatenate([jnp.zeros((1,), jnp.int32), pcum[:-1]])
    within = jnp.sum(onehot * (cum - 1), axis=1)
    baseof = jnp.sum(onehot * pbase[None, :], axis=1)
    dest = baseof + within

    tile_ends = pcum // _TG
    t = jnp.arange(p // _TG, dtype=jnp.int32)
    eid_tile = jnp.searchsorted(tile_ends, t, side="right").astype(jnp.int32)
    valid = (eid_tile < _E).astype(jnp.int32)
    eid_tile = jnp.minimum(eid_tile, _E - 1)

    dest2 = dest.reshape(nk // _CH, _CH)
    tok2 = (jnp.arange(nk, dtype=jnp.int32) // _K).reshape(nk // _CH, _CH)
    p01 = dest.reshape(n, _K)
    p0_2d = p01[:, 0].reshape(n // _CH, _CH)
    p1_2d = p01[:, 1].reshape(n // _CH, _CH)

    xg = _make_dispatch(n, d, p)(flat, tok2, dest2)
    y = _gmm(eid_tile, valid, xg, W1, W2)
    y0, y1 = _make_gather2(n, d, p)(y, p0_2d, p1_2d)
    out = _ln(y0, y1, gsc[:, 0].reshape(n, 1), gsc[:, 1].reshape(n, 1),
              flat, ln_g, ln_b)
    return out.reshape(s, b, d), bias
